# Initial kernel scaffold; baseline (speedup 1.0000x reference)
import jax
import jax.numpy as jnp
from jax.experimental import pallas as pl


def kernel(action, table):
    raise NotImplementedError("write your pallas kernel here")


# speedup vs baseline: 15.0295x; 15.0295x over previous
import jax
import jax.numpy as jnp
from jax import lax
from jax.experimental import pallas as pl
from jax.experimental.pallas import tpu as pltpu
from jax.experimental.pallas import tpu_sc as plsc

NUM_ACTIONS = 6
ACTION_DIM = 32
BATCH = 16384
HIST = 200

NC = 2
NS = 16
NW = NC * NS
L = 16

TUP = 4
ROW_T = HIST // TUP
G = 8
CHUNK_A = G * HIST
CHUNK_T = G * ROW_T
NCHUNKS = BATCH // G
CPW = NCHUNKS // NW
TVECS = CHUNK_T // L


def _sc_body(act_hbm, ptab_hbm, out_hbm, act_v, rows_v, sem):
    wid = lax.axis_index("s") * NC + lax.axis_index("c")
    base = wid * CPW
    i16 = lax.iota(jnp.int32, 16)

    @pl.loop(0, CPW)
    def _chunk(c):
        chunk = base + c
        pltpu.sync_copy(act_hbm.at[chunk], act_v)
        copies = []
        for t in range(TVECS):
            pos = i16 * TUP + t * (L * TUP)
            a0 = plsc.load_gather(act_v, [pos])
            a1 = plsc.load_gather(act_v, [pos + 1])
            a2 = plsc.load_gather(act_v, [pos + 2])
            a3 = plsc.load_gather(act_v, [pos + 3])
            idx = ((a0 * NUM_ACTIONS + a1) * NUM_ACTIONS + a2) * NUM_ACTIONS + a3
            copies.append(
                pltpu.async_copy(
                    ptab_hbm.at[idx], rows_v.at[pl.ds(t * L, L)], sem
                )
            )
        for cp in copies:
            cp.wait()
        pltpu.sync_copy(rows_v, out_hbm.at[chunk])


def kernel(action, table):
    aidx = jnp.arange(NUM_ACTIONS**TUP, dtype=jnp.int32)
    parts = []
    for k in range(TUP):
        ak = (aidx // (NUM_ACTIONS ** (TUP - 1 - k))) % NUM_ACTIONS
        parts.append(jnp.take(table, ak, axis=0))
    ptab = jnp.concatenate(parts, axis=1)

    act2 = action.reshape(NCHUNKS, CHUNK_A)
    kfn = pl.kernel(
        _sc_body,
        out_type=jax.ShapeDtypeStruct(
            (NCHUNKS, CHUNK_T, TUP * ACTION_DIM), jnp.float32
        ),
        mesh=plsc.VectorSubcoreMesh(core_axis_name="c", subcore_axis_name="s"),
        compiler_params=pltpu.CompilerParams(needs_layout_passes=False),
        scratch_types=[
            pltpu.VMEM((CHUNK_A,), jnp.int32),
            pltpu.VMEM((CHUNK_T, TUP * ACTION_DIM), jnp.float32),
            pltpu.SemaphoreType.DMA,
        ],
    )
    out3 = kfn(act2, ptab)
    return out3.reshape(BATCH, HIST * ACTION_DIM)


# speedup vs baseline: 15.5453x; 1.0343x over previous
import jax
import jax.numpy as jnp
from jax import lax
from jax.experimental import pallas as pl
from jax.experimental.pallas import tpu as pltpu
from jax.experimental.pallas import tpu_sc as plsc

NUM_ACTIONS = 6
ACTION_DIM = 32
BATCH = 16384
HIST = 200

NC = 2
NS = 16
NW = NC * NS
L = 16

TUP = 4
ROW_T = HIST // TUP
G = 8
CHUNK_A = G * HIST
CHUNK_T = G * ROW_T
ROW_W = TUP * ACTION_DIM
NCHUNKS = BATCH // G
CPW = NCHUNKS // NW
TVECS = CHUNK_T // L


def _sc_body(act_hbm, ptab_hbm, out_hbm, a0_v, a1_v, r0_v, r1_v,
             is0, is1, gs0, gs1, ws0, ws1):
    wid = lax.axis_index("s") * NC + lax.axis_index("c")
    base = wid * CPW
    i16 = lax.iota(jnp.int32, 16)
    acts, rows = (a0_v, a1_v), (r0_v, r1_v)
    isem, gsem, wsem = (is0, is1), (gs0, gs1), (ws0, ws1)

    def fire_idx(i, b):
        pltpu.async_copy(act_hbm.at[base + i], acts[b], isem[b])

    def drain_idx(b):
        pltpu.make_async_copy(act_hbm.at[0], acts[b], isem[b]).wait()

    def fire_gathers(b):
        for t in range(TVECS):
            pos = i16 * TUP + t * (L * TUP)
            a0 = plsc.load_gather(acts[b], [pos])
            a1 = plsc.load_gather(acts[b], [pos + 1])
            a2 = plsc.load_gather(acts[b], [pos + 2])
            a3 = plsc.load_gather(acts[b], [pos + 3])
            idx = ((a0 * NUM_ACTIONS + a1) * NUM_ACTIONS + a2) * NUM_ACTIONS + a3
            pltpu.async_copy(
                ptab_hbm.at[idx], rows[b].at[pl.ds(t * L, L)], gsem[b]
            )

    def drain_gathers(b):
        pltpu.make_async_copy(out_hbm.at[0], rows[b], gsem[b]).wait()

    def fire_write(i, b):
        pltpu.async_copy(rows[b], out_hbm.at[base + i], wsem[b])

    def drain_write(b):
        pltpu.make_async_copy(out_hbm.at[0], rows[b], wsem[b]).wait()

    def slot(i, b, first, last):
        @pl.when(jnp.logical_not(first))
        def _():
            drain_write(b)
        drain_idx(b)
        fire_gathers(b)
        @pl.when(jnp.logical_not(last))
        def _():
            fire_idx(i + 2, b)
        @pl.when(i > 0)
        def _():
            drain_gathers(1 - b)
            fire_write(i - 1, 1 - b)

    fire_idx(0, 0)
    fire_idx(1, 1)

    @pl.loop(0, CPW, step=2)
    def _pair(c0):
        slot(c0, 0, c0 == 0, c0 + 2 >= CPW)
        slot(c0 + 1, 1, c0 == 0, c0 + 3 >= CPW)

    drain_gathers((CPW - 1) % 2)
    fire_write(CPW - 1, (CPW - 1) % 2)
    drain_write(0)
    drain_write(1)


def kernel(action, table):
    aidx = jnp.arange(NUM_ACTIONS**TUP, dtype=jnp.int32)
    parts = []
    for k in range(TUP):
        ak = (aidx // (NUM_ACTIONS ** (TUP - 1 - k))) % NUM_ACTIONS
        parts.append(jnp.take(table, ak, axis=0))
    ptab = jnp.concatenate(parts, axis=1)

    act2 = action.reshape(NCHUNKS, CHUNK_A)
    kfn = pl.kernel(
        _sc_body,
        out_type=jax.ShapeDtypeStruct((NCHUNKS, CHUNK_T, ROW_W), jnp.float32),
        mesh=plsc.VectorSubcoreMesh(core_axis_name="c", subcore_axis_name="s"),
        compiler_params=pltpu.CompilerParams(needs_layout_passes=False),
        scratch_types=[
            pltpu.VMEM((CHUNK_A,), jnp.int32),
            pltpu.VMEM((CHUNK_A,), jnp.int32),
            pltpu.VMEM((CHUNK_T, ROW_W), jnp.float32),
            pltpu.VMEM((CHUNK_T, ROW_W), jnp.float32),
            pltpu.SemaphoreType.DMA,
            pltpu.SemaphoreType.DMA,
            pltpu.SemaphoreType.DMA,
            pltpu.SemaphoreType.DMA,
            pltpu.SemaphoreType.DMA,
            pltpu.SemaphoreType.DMA,
        ],
    )
    out3 = kfn(act2, ptab)
    return out3.reshape(BATCH, HIST * ACTION_DIM)
